# baseline (speedup 1.0000x reference)
import functools

import jax
import jax.numpy as jnp
from jax import lax
from jax.experimental import pallas as pl
from jax.experimental.pallas import tpu as pltpu
from jax.experimental.pallas import tpu_sc as plsc

_F = 32
_BI = 8192
_Q = _BI // 4


def _tc_relayout_body(xt_ref, out_ref):
    x = xt_ref[...]
    x_hi = lax.bitcast_convert_type(
        lax.bitcast_convert_type(x, jnp.int32) & jnp.int32(-65536),
        jnp.float32,
    )
    x_lo = x - x_hi
    fi = lax.broadcasted_iota(jnp.int32, (_F, 128), 0)
    li = lax.broadcasted_iota(jnp.int32, (_F, 128), 1)
    acc = None
    for k in range(4):
        ek = jnp.where(li == fi + _F * k, 1.0, 0.0)
        for piece in (x_hi, x_lo):
            part = lax.dot_general(
                piece[:, _Q * k:_Q * (k + 1)], ek,
                (((0,), (0,)), ((), ())),
                preferred_element_type=jnp.float32,
            )
            acc = part if acc is None else acc + part
    out_ref[...] = acc


@functools.lru_cache(maxsize=None)
def _make_tc_relayout(V):
    n_blocks = pl.cdiv(V, _BI)
    return pl.pallas_call(
        _tc_relayout_body,
        grid=(n_blocks,),
        in_specs=[pl.BlockSpec((_F, _BI), lambda i: (0, i))],
        out_specs=pl.BlockSpec((_Q, 128), lambda i: (i, 0)),
        out_shape=jax.ShapeDtypeStruct((n_blocks * _Q, 128), jnp.float32),
    )


_BB = 128


def _tc_out_body(y_ref, out_ref):
    y = y_ref[...]
    y_hi = lax.bitcast_convert_type(
        lax.bitcast_convert_type(y, jnp.int32) & jnp.int32(-65536),
        jnp.float32,
    )
    y_lo = y - y_hi
    ii = lax.broadcasted_iota(jnp.int32, (_F, _F), 0)
    jj = lax.broadcasted_iota(jnp.int32, (_F, _F), 1)
    eye = jnp.where(ii == jj, 1.0, 0.0)
    ti = lax.broadcasted_iota(jnp.int32, (_F, 128), 0)
    li = lax.broadcasted_iota(jnp.int32, (_F, 128), 1)
    acc = None
    for j in range(4):
        pj = jnp.where(li == 4 * ti + j, 1.0, 0.0)
        for piece in (y_hi, y_lo):
            sl = piece[:, j * 832:(j + 1) * 832]
            t1 = lax.dot_general(
                sl, eye, (((0,), (0,)), ((), ())),
                preferred_element_type=jnp.float32,
            )
            t2 = lax.dot_general(
                t1, pj, (((1,), (0,)), ((), ())),
                preferred_element_type=jnp.float32,
            )
            acc = t2 if acc is None else acc + t2
    out_ref[...] = acc.reshape(26, _F, 128)


@functools.lru_cache(maxsize=None)
def _make_tc_out(R, C):
    n_blocks = R // _BB
    return pl.pallas_call(
        _tc_out_body,
        grid=(n_blocks,),
        in_specs=[pl.BlockSpec((_F, C * _F * 4), lambda i: (i, 0))],
        out_specs=pl.BlockSpec((C, _F, _BB), lambda i: (0, 0, i)),
        out_shape=jax.ShapeDtypeStruct((C, _F, R), jnp.float32),
    )


@functools.lru_cache(maxsize=None)
def _make_lookup(B, n_workers, chunk, table_rows):
    b_per_w = B // n_workers
    n_chunks = b_per_w // chunk
    mesh = plsc.VectorSubcoreMesh(core_axis_name="c", subcore_axis_name="s")

    @functools.partial(
        pl.kernel,
        mesh=mesh,
        out_type=jax.ShapeDtypeStruct((B, _F), jnp.float32),
        scratch_types=[
            pltpu.VMEM((2, chunk), jnp.int32),
            pltpu.VMEM((2, chunk, _F), jnp.float32),
            pltpu.SemaphoreType.DMA,
            pltpu.SemaphoreType.DMA,
        ],
        compiler_params=pltpu.CompilerParams(use_tc_tiling_on_sc=False),
    )
    def lookup(idx_hbm, table_hbm, out_hbm, idx_v, rows_v, sem_g, sem_o):
        wid = lax.axis_index("s") * 2 + lax.axis_index("c")
        base = wid * b_per_w

        def remap(slot):
            ref = idx_v.at[slot]
            for b in range(chunk // 16):
                sl = pl.ds(b * 16, 16)
                t = ref[sl]
                j = (
                    ((t >> 13) << 13)
                    + ((t & 2047) << 2)
                    + ((t >> 11) & 3)
                )
                ref[sl] = j

        def start_gather(slot):
            return pltpu.async_copy(
                table_hbm.at[idx_v.at[slot]], rows_v.at[slot], sem_g
            )

        pltpu.sync_copy(idx_hbm.at[pl.ds(base, chunk)], idx_v.at[0])
        remap(0)
        gathers = [start_gather(0)]
        outs = [None, None]
        for g in range(n_chunks):
            s = g % 2
            ns = (g + 1) % 2
            if g + 1 < n_chunks:
                pltpu.sync_copy(
                    idx_hbm.at[pl.ds(base + (g + 1) * chunk, chunk)],
                    idx_v.at[ns],
                )
                remap(ns)
            gathers[g].wait()
            if g + 1 < n_chunks:
                if outs[ns] is not None:
                    outs[ns].wait()
                gathers.append(start_gather(ns))
            outs[s] = pltpu.async_copy(
                rows_v.at[s], out_hbm.at[pl.ds(base + g * chunk, chunk)], sem_o
            )
        if n_chunks >= 2:
            outs[(n_chunks - 2) % 2].wait()
        outs[(n_chunks - 1) % 2].wait()

    return lookup


def kernel(inputs, embedding):
    V, F = embedding.shape
    B = inputs.shape[0] * inputs.shape[1]
    flat_idx = inputs.reshape(B)
    table_lin = _make_tc_relayout(V)(embedding.T)
    table_rows = table_lin.shape[0] * (128 // F)
    table32 = table_lin.reshape(table_rows, F)
    out = _make_lookup(B, 32, 1664, table_rows)(flat_idx, table32)
    R, C = inputs.shape
    y = out.reshape(R // 4, C * F * 4)
    out3 = _make_tc_out(R, C)(y)
    return jnp.transpose(out3, (2, 0, 1))


# speedup vs baseline: 1.2189x; 1.2189x over previous
import functools

import jax
import jax.numpy as jnp
from jax import lax
from jax.experimental import pallas as pl
from jax.experimental.pallas import tpu as pltpu
from jax.experimental.pallas import tpu_sc as plsc

_F = 32
_BI = 16384
_Q = _BI // 4
_BB = 256


def _split_hi_lo(x):
    hi = lax.bitcast_convert_type(
        lax.bitcast_convert_type(x, jnp.int32) & jnp.int32(-65536),
        jnp.float32,
    )
    return hi, x - hi


def _tc_table_body(xt_ref, out_ref):
    x = xt_ref[...]
    x_hi, x_lo = _split_hi_lo(x)
    fi = lax.broadcasted_iota(jnp.int32, (_F, 128), 0)
    li = lax.broadcasted_iota(jnp.int32, (_F, 128), 1)
    acc = None
    for k in range(4):
        ek = jnp.where(li == fi + _F * k, 1.0, 0.0)
        for piece in (x_hi, x_lo):
            part = lax.dot_general(
                piece[:, _Q * k:_Q * (k + 1)], ek,
                (((0,), (0,)), ((), ())),
                preferred_element_type=jnp.float32,
            )
            acc = part if acc is None else acc + part
    out_ref[...] = acc


@functools.lru_cache(maxsize=None)
def _make_tc_table(V):
    n_blocks = pl.cdiv(V, _BI)
    return pl.pallas_call(
        _tc_table_body,
        grid=(n_blocks,),
        in_specs=[pl.BlockSpec((_F, _BI), lambda i: (0, i))],
        out_specs=pl.BlockSpec((_Q, 128), lambda i: (i, 0)),
        out_shape=jax.ShapeDtypeStruct((n_blocks * _Q, 128), jnp.float32),
    )


def _tc_out_body(y_ref, out_ref):
    y = y_ref[...]
    y_hi, y_lo = _split_hi_lo(y)
    ti = lax.broadcasted_iota(jnp.int32, (_BB // 4, _BB), 0)
    li = lax.broadcasted_iota(jnp.int32, (_BB // 4, _BB), 1)
    acc = None
    for j in range(4):
        pj = jnp.where(li == 4 * ti + j, 1.0, 0.0)
        for piece in (y_hi, y_lo):
            sl = piece[:, j * 832:(j + 1) * 832]
            t2 = lax.dot_general(
                sl, pj, (((0,), (0,)), ((), ())),
                preferred_element_type=jnp.float32,
            )
            acc = t2 if acc is None else acc + t2
    out_ref[...] = acc.reshape(26, _F, _BB)


@functools.lru_cache(maxsize=None)
def _make_tc_out(R, C):
    n_blocks = R // _BB
    return pl.pallas_call(
        _tc_out_body,
        grid=(n_blocks,),
        in_specs=[pl.BlockSpec((_BB // 4, C * _F * 4), lambda i: (i, 0))],
        out_specs=pl.BlockSpec((C, _F, _BB), lambda i: (0, 0, i)),
        out_shape=jax.ShapeDtypeStruct((C, _F, R), jnp.float32),
    )


@functools.lru_cache(maxsize=None)
def _make_lookup(B, n_workers, chunk, table_rows):
    b_per_w = B // n_workers
    n_chunks = b_per_w // chunk
    mesh = plsc.VectorSubcoreMesh(core_axis_name="c", subcore_axis_name="s")

    @functools.partial(
        pl.kernel,
        mesh=mesh,
        out_type=jax.ShapeDtypeStruct((B, _F), jnp.float32),
        scratch_types=[
            pltpu.VMEM((2, chunk), jnp.int32),
            pltpu.VMEM((2, chunk, _F), jnp.float32),
            pltpu.SemaphoreType.DMA,
            pltpu.SemaphoreType.DMA,
        ],
        compiler_params=pltpu.CompilerParams(use_tc_tiling_on_sc=False),
    )
    def lookup(idx_hbm, table_hbm, out_hbm, idx_v, rows_v, sem_g, sem_o):
        wid = lax.axis_index("s") * 2 + lax.axis_index("c")
        base = wid * b_per_w

        def remap(slot):
            ref = idx_v.at[slot]
            for b in range(chunk // 16):
                sl = pl.ds(b * 16, 16)
                t = ref[sl]
                ref[sl] = (
                    ((t >> 14) << 14)
                    + ((t & 4095) << 2)
                    + ((t >> 12) & 3)
                )

        def start_gather(slot):
            return pltpu.async_copy(
                table_hbm.at[idx_v.at[slot]], rows_v.at[slot], sem_g
            )

        pltpu.sync_copy(idx_hbm.at[pl.ds(base, chunk)], idx_v.at[0])
        remap(0)
        gathers = [start_gather(0)]
        outs = [None, None]
        for g in range(n_chunks):
            s = g % 2
            ns = (g + 1) % 2
            if g + 1 < n_chunks:
                pltpu.sync_copy(
                    idx_hbm.at[pl.ds(base + (g + 1) * chunk, chunk)],
                    idx_v.at[ns],
                )
                remap(ns)
            gathers[g].wait()
            if g + 1 < n_chunks:
                if outs[ns] is not None:
                    outs[ns].wait()
                gathers.append(start_gather(ns))
            outs[s] = pltpu.async_copy(
                rows_v.at[s], out_hbm.at[pl.ds(base + g * chunk, chunk)],
                sem_o,
            )
        if n_chunks >= 2:
            outs[(n_chunks - 2) % 2].wait()
        outs[(n_chunks - 1) % 2].wait()

    return lookup


def kernel(inputs, embedding):
    V, F = embedding.shape
    R, C = inputs.shape
    B = R * C
    flat_idx = inputs.reshape(B)
    table_lin = _make_tc_table(V)(embedding.T)
    table_rows = table_lin.shape[0] * (128 // F)
    table32 = table_lin.reshape(table_rows, F)
    out = _make_lookup(B, 32, 1664, table_rows)(flat_idx, table32)
    y = out.reshape(R // 4, C * F * 4)
    out3 = _make_tc_out(R, C)(y)
    return jnp.transpose(out3, (2, 0, 1))
